# baseline (speedup 1.0000x reference)
import functools

import jax
import jax.numpy as jnp
from jax import lax
from jax.experimental import pallas as pl
from jax.experimental.pallas import tpu as pltpu
from jax.experimental.pallas import tpu_sc as plsc

_B = 16384
_L = 20
_MID = 128
_NC = 2
_NS = 16
_NW = _NC * _NS
_BPW = _B // _NW
_GRP = _BPW // 16

_T1P, _T2P, _T3P = 128, 3072, 1024


def _prep_body(states_ref, emb1_ref, emb2_ref, emb3_ref, w1_ref, b1_ref,
               w2_ref, b2_ref, base_ref, t1_ref, t2_ref, t3a_ref, t3d_ref,
               t3e_ref):
    hi = lax.Precision.HIGHEST
    w2 = w2_ref[...]
    v = lax.dot_general(w2, w1_ref[...], (((1,), (0,)), ((), ())),
                        precision=hi)
    c = jnp.sum(b1_ref[...] * w2) + b2_ref[0, 0]

    def projT(vk, emb, pad):
        row = lax.dot_general(vk, emb, (((1,), (1,)), ((), ())),
                              precision=hi)
        if pad:
            row = jnp.concatenate(
                [row, jnp.zeros((1, pad), jnp.float32)], axis=1)
        return row

    base_ref[...] = projT(v[:, 0:128], states_ref[...], 0) + c
    t1_ref[...] = projT(v[:, 128:256], emb1_ref[...], _T1P - 5)
    t2_ref[...] = projT(v[:, 256:384], emb2_ref[...], _T2P - 3000)
    t3a_ref[...] = projT(v[:, 384:512], emb3_ref[...], _T3P - 1000)
    t3d_ref[...] = projT(v[:, 512:640], emb3_ref[...], _T3P - 1000)
    t3e_ref[...] = projT(v[:, 640:768], emb3_ref[...], _T3P - 1000)


_prep = pl.pallas_call(
    _prep_body,
    out_shape=[
        jax.ShapeDtypeStruct((1, _B), jnp.float32),
        jax.ShapeDtypeStruct((1, _T1P), jnp.float32),
        jax.ShapeDtypeStruct((1, _T2P), jnp.float32),
        jax.ShapeDtypeStruct((1, _T3P), jnp.float32),
        jax.ShapeDtypeStruct((1, _T3P), jnp.float32),
        jax.ShapeDtypeStruct((1, _T3P), jnp.float32),
    ],
)


def _make_sc_gather():
    mesh = plsc.VectorSubcoreMesh(core_axis_name="c", subcore_axis_name="s")

    @functools.partial(
        pl.kernel,
        mesh=mesh,
        out_type=jax.ShapeDtypeStruct((_B,), jnp.float32),
        compiler_params=pltpu.CompilerParams(
            needs_layout_passes=False, use_tc_tiling_on_sc=False),
        scratch_types=[
            pltpu.VMEM((_BPW, _L), jnp.int32),
            pltpu.VMEM((_BPW, _L), jnp.int32),
            pltpu.VMEM((_BPW, _L), jnp.int32),
            pltpu.VMEM((_BPW, _L), jnp.int32),
            pltpu.VMEM((_BPW, _L), jnp.int32),
            pltpu.VMEM((_T1P,), jnp.float32),
            pltpu.VMEM((_T2P,), jnp.float32),
            pltpu.VMEM((_T3P,), jnp.float32),
            pltpu.VMEM((_T3P,), jnp.float32),
            pltpu.VMEM((_T3P,), jnp.float32),
            pltpu.VMEM((_BPW,), jnp.float32),
            pltpu.VMEM((_BPW,), jnp.float32),
        ],
    )
    def sc_k(ac_hbm, play_hbm, atk_hbm, dfd_hbm, evo_hbm, base_hbm,
             t1_hbm, t2_hbm, t3a_hbm, t3d_hbm, t3e_hbm, out_hbm,
             ac_v, play_v, atk_v, dfd_v, evo_v,
             t1_v, t2_v, t3a_v, t3d_v, t3e_v, base_v, out_v):
        wid = lax.axis_index("s") * _NC + lax.axis_index("c")
        b0 = wid * _BPW
        pltpu.sync_copy(ac_hbm.at[pl.ds(b0, _BPW)], ac_v)
        pltpu.sync_copy(play_hbm.at[pl.ds(b0, _BPW)], play_v)
        pltpu.sync_copy(atk_hbm.at[pl.ds(b0, _BPW)], atk_v)
        pltpu.sync_copy(dfd_hbm.at[pl.ds(b0, _BPW)], dfd_v)
        pltpu.sync_copy(evo_hbm.at[pl.ds(b0, _BPW)], evo_v)
        pltpu.sync_copy(t1_hbm.at[pl.ds(0, _T1P)], t1_v)
        pltpu.sync_copy(t2_hbm.at[pl.ds(0, _T2P)], t2_v)
        pltpu.sync_copy(t3a_hbm.at[pl.ds(0, _T3P)], t3a_v)
        pltpu.sync_copy(t3d_hbm.at[pl.ds(0, _T3P)], t3d_v)
        pltpu.sync_copy(t3e_hbm.at[pl.ds(0, _T3P)], t3e_v)
        pltpu.sync_copy(base_hbm.at[pl.ds(b0, _BPW)], base_v)

        lane = lax.iota(jnp.int32, 16)

        def group(g, carry):
            svec = g * 16 + lane
            a1 = base_v[pl.ds(g * 16, 16)]
            a2 = a1 - a1
            a3 = a2
            a4 = a2
            a5 = a2
            for l in range(_L):
                lvec = lane - lane + l
                a1 = a1 + plsc.load_gather(
                    t1_v, [plsc.load_gather(ac_v, [svec, lvec])])
                a2 = a2 + plsc.load_gather(
                    t2_v, [plsc.load_gather(play_v, [svec, lvec])])
                a3 = a3 + plsc.load_gather(
                    t3a_v, [plsc.load_gather(atk_v, [svec, lvec])])
                a4 = a4 + plsc.load_gather(
                    t3d_v, [plsc.load_gather(dfd_v, [svec, lvec])])
                a5 = a5 + plsc.load_gather(
                    t3e_v, [plsc.load_gather(evo_v, [svec, lvec])])
            out_v[pl.ds(g * 16, 16)] = (a1 + a2) + (a3 + a4) + a5
            return carry

        lax.fori_loop(0, _GRP, group, 0)
        pltpu.sync_copy(out_v, out_hbm.at[pl.ds(b0, _BPW)])

    return sc_k


_sc_gather = _make_sc_gather()


def kernel(states, action_categories, play_card_ids, attacking_card_ids,
           attacked_card_ids, evolving_card_ids, emb1, emb2, emb3,
           W1, b1, W2, b2):
    base, t1, t2, t3a, t3d, t3e = _prep(
        states, emb1, emb2, emb3, W1, b1.reshape(1, _MID), W2,
        b2.reshape(1, 1))
    i32 = jnp.int32
    out = _sc_gather(
        action_categories.astype(i32),
        play_card_ids.astype(i32),
        attacking_card_ids.astype(i32),
        attacked_card_ids.astype(i32),
        evolving_card_ids.astype(i32),
        base.reshape(-1),
        t1.reshape(-1), t2.reshape(-1),
        t3a.reshape(-1), t3d.reshape(-1), t3e.reshape(-1))
    return out.reshape(_B, 1)


# speedup vs baseline: 1.2924x; 1.2924x over previous
import functools

import jax
import jax.numpy as jnp
from jax import lax
from jax.experimental import pallas as pl
from jax.experimental.pallas import tpu as pltpu
from jax.experimental.pallas import tpu_sc as plsc

_B = 16384
_L = 20
_MID = 128
_NC = 2
_NS = 16
_NW = _NC * _NS
_BPW = _B // _NW
_C = 128
_NCHUNK = _BPW // _C
_GC = _C // 16

_T1P, _T2P, _T3P = 128, 3072, 1024


def _prep_body(ac_ref, play_ref, atk_ref, dfd_ref, evo_ref, states_ref,
               emb1_ref, emb2_ref, emb3_ref, w1_ref, b1_ref, w2_ref, b2_ref,
               packed_ref, base_ref, t1_ref, t2_ref, t3a_ref, t3d_ref,
               t3e_ref):
    packed_ref[...] = jnp.concatenate(
        [ac_ref[...], play_ref[...], atk_ref[...], dfd_ref[...], evo_ref[...],
         jnp.zeros((1024, 128 - 5 * _L), jnp.int32)], axis=1)

    @pl.when(pl.program_id(0) == 0)
    def _():
        hi = lax.Precision.HIGHEST
        w2 = w2_ref[...]
        v = lax.dot_general(w2, w1_ref[...], (((1,), (0,)), ((), ())),
                            precision=hi)
        c = jnp.sum(b1_ref[...] * w2) + b2_ref[0, 0]

        def projT(vk, emb, pad):
            row = lax.dot_general(vk, emb, (((1,), (1,)), ((), ())),
                                  precision=hi)
            if pad:
                row = jnp.concatenate(
                    [row, jnp.zeros((1, pad), jnp.float32)], axis=1)
            return row

        base_ref[...] = projT(v[:, 0:128], states_ref[...], 0) + c
        t1_ref[...] = projT(v[:, 128:256], emb1_ref[...], _T1P - 5)
        t2_ref[...] = projT(v[:, 256:384], emb2_ref[...], _T2P - 3000)
        t3a_ref[...] = projT(v[:, 384:512], emb3_ref[...], _T3P - 1000)
        t3d_ref[...] = projT(v[:, 512:640], emb3_ref[...], _T3P - 1000)
        t3e_ref[...] = projT(v[:, 640:768], emb3_ref[...], _T3P - 1000)


_prep = pl.pallas_call(
    _prep_body,
    grid=(16,),
    in_specs=[
        pl.BlockSpec((1024, _L), lambda i: (i, 0)),
        pl.BlockSpec((1024, _L), lambda i: (i, 0)),
        pl.BlockSpec((1024, _L), lambda i: (i, 0)),
        pl.BlockSpec((1024, _L), lambda i: (i, 0)),
        pl.BlockSpec((1024, _L), lambda i: (i, 0)),
        pl.BlockSpec((_B, 128), lambda i: (0, 0)),
        pl.BlockSpec((5, 128), lambda i: (0, 0)),
        pl.BlockSpec((3000, 128), lambda i: (0, 0)),
        pl.BlockSpec((1000, 128), lambda i: (0, 0)),
        pl.BlockSpec((128, 768), lambda i: (0, 0)),
        pl.BlockSpec((1, 128), lambda i: (0, 0)),
        pl.BlockSpec((1, 128), lambda i: (0, 0)),
        pl.BlockSpec((1, 1), lambda i: (0, 0)),
    ],
    out_specs=[
        pl.BlockSpec((1024, 128), lambda i: (i, 0)),
        pl.BlockSpec((1, _B), lambda i: (0, 0)),
        pl.BlockSpec((1, _T1P), lambda i: (0, 0)),
        pl.BlockSpec((1, _T2P), lambda i: (0, 0)),
        pl.BlockSpec((1, _T3P), lambda i: (0, 0)),
        pl.BlockSpec((1, _T3P), lambda i: (0, 0)),
        pl.BlockSpec((1, _T3P), lambda i: (0, 0)),
    ],
    out_shape=[
        jax.ShapeDtypeStruct((_B, 128), jnp.int32),
        jax.ShapeDtypeStruct((1, _B), jnp.float32),
        jax.ShapeDtypeStruct((1, _T1P), jnp.float32),
        jax.ShapeDtypeStruct((1, _T2P), jnp.float32),
        jax.ShapeDtypeStruct((1, _T3P), jnp.float32),
        jax.ShapeDtypeStruct((1, _T3P), jnp.float32),
        jax.ShapeDtypeStruct((1, _T3P), jnp.float32),
    ],
)


def _make_sc_gather():
    mesh = plsc.VectorSubcoreMesh(core_axis_name="c", subcore_axis_name="s")

    @functools.partial(
        pl.kernel,
        mesh=mesh,
        out_type=jax.ShapeDtypeStruct((_B,), jnp.float32),
        compiler_params=pltpu.CompilerParams(needs_layout_passes=False),
        scratch_types=[
            pltpu.VMEM((_C * 128,), jnp.int32),
            pltpu.VMEM((_C * 128,), jnp.int32),
            pltpu.VMEM((_T1P,), jnp.float32),
            pltpu.VMEM((_T2P,), jnp.float32),
            pltpu.VMEM((_T3P,), jnp.float32),
            pltpu.VMEM((_T3P,), jnp.float32),
            pltpu.VMEM((_T3P,), jnp.float32),
            pltpu.VMEM((_BPW,), jnp.float32),
            pltpu.VMEM((_BPW,), jnp.float32),
            pltpu.SemaphoreType.DMA,
            pltpu.SemaphoreType.DMA,
        ],
    )
    def sc_k(idx_hbm, base_hbm, t1_hbm, t2_hbm, t3a_hbm, t3d_hbm, t3e_hbm,
             out_hbm,
             ib0, ib1, t1_v, t2_v, t3a_v, t3d_v, t3e_v, base_v, out_v,
             s0, s1):
        wid = lax.axis_index("s") * _NC + lax.axis_index("c")
        b0 = wid * _BPW
        bufs = (ib0, ib1)
        sems = (s0, s1)

        def start(j):
            return pltpu.async_copy(
                idx_hbm.at[pl.ds((b0 + j * _C) * 128, _C * 128)],
                bufs[j % 2], sems[j % 2])

        cps = [start(0), start(1)]
        pltpu.sync_copy(t1_hbm, t1_v)
        pltpu.sync_copy(t2_hbm, t2_v)
        pltpu.sync_copy(t3a_hbm, t3a_v)
        pltpu.sync_copy(t3d_hbm, t3d_v)
        pltpu.sync_copy(t3e_hbm, t3e_v)
        pltpu.sync_copy(base_hbm.at[pl.ds(b0, _BPW)], base_v)

        lane128 = lax.iota(jnp.int32, 16) * 128

        for ch in range(_NCHUNK):
            buf = bufs[ch % 2]
            cps[ch % 2].wait()

            def group(gi, carry):
                a0 = gi * (16 * 128)
                a1 = base_v[pl.ds(ch * _C + gi * 16, 16)]
                a2 = a1 - a1
                a3 = a2
                a4 = a2
                a5 = a2
                for l in range(_L):
                    base_a = lane128 + (a0 + l)
                    a1 = a1 + plsc.load_gather(
                        t1_v, [plsc.load_gather(buf, [base_a])])
                    a2 = a2 + plsc.load_gather(
                        t2_v, [plsc.load_gather(buf, [base_a + _L])])
                    a3 = a3 + plsc.load_gather(
                        t3a_v, [plsc.load_gather(buf, [base_a + 2 * _L])])
                    a4 = a4 + plsc.load_gather(
                        t3d_v, [plsc.load_gather(buf, [base_a + 3 * _L])])
                    a5 = a5 + plsc.load_gather(
                        t3e_v, [plsc.load_gather(buf, [base_a + 4 * _L])])
                out_v[pl.ds(ch * _C + gi * 16, 16)] = (
                    (a1 + a2) + (a3 + a4) + a5)
                return carry

            lax.fori_loop(0, _GC, group, 0)
            if ch + 2 < _NCHUNK:
                cps[ch % 2] = start(ch + 2)

        pltpu.sync_copy(out_v, out_hbm.at[pl.ds(b0, _BPW)])

    return sc_k


_sc_gather = _make_sc_gather()


def kernel(states, action_categories, play_card_ids, attacking_card_ids,
           attacked_card_ids, evolving_card_ids, emb1, emb2, emb3,
           W1, b1, W2, b2):
    i32 = jnp.int32
    packed, base, t1, t2, t3a, t3d, t3e = _prep(
        action_categories.astype(i32), play_card_ids.astype(i32),
        attacking_card_ids.astype(i32), attacked_card_ids.astype(i32),
        evolving_card_ids.astype(i32),
        states, emb1, emb2, emb3, W1, b1.reshape(1, _MID), W2,
        b2.reshape(1, 1))
    out = _sc_gather(
        packed.reshape(-1), base.reshape(-1),
        t1.reshape(-1), t2.reshape(-1),
        t3a.reshape(-1), t3d.reshape(-1), t3e.reshape(-1))
    return out.reshape(_B, 1)
